# Initial kernel scaffold; baseline (speedup 1.0000x reference)
import jax
import jax.numpy as jnp
from jax.experimental import pallas as pl


def kernel(x, edge_index, W1_l, b1, W1_r, W2_l, b2, W2_r):
    raise NotImplementedError("write your pallas kernel here")


# speedup vs baseline: 3.8510x; 3.8510x over previous
import functools

import jax
import jax.numpy as jnp
from jax import lax
from jax.experimental import pallas as pl
from jax.experimental.pallas import tpu as pltpu
from jax.experimental.pallas import tpu_sc as plsc

N = 10000
E = 320000
F = 128
OUT_F = 121

NC = 2
NS = 16
NW = NC * NS
EPW = E // NW
CH = 80
NCHUNK = EPW // CH
G = NW * NCHUNK
NPAD = 10240
RPT = NPAD // NS
CW = 16

_mesh = plsc.VectorSubcoreMesh(core_axis_name="c", subcore_axis_name="s")


@functools.partial(
    pl.kernel,
    mesh=_mesh,
    out_type=[
        jax.ShapeDtypeStruct((NC, NPAD, F), jnp.float32),
    ],
    scratch_types=[
        pltpu.VMEM((CH,), jnp.int32),
        pltpu.VMEM((CH,), jnp.int32),
        pltpu.VMEM((CH, F), jnp.float32),
        pltpu.VMEM_SHARED((NPAD, F), jnp.float32),
        pltpu.SemaphoreType.DMA,
    ],
)
def _sc_agg(x_hbm, src_hbm, dst_hbm, zf_hbm,
            sums_out,
            src_f, dst_f, rows_v, acc_sh, sem):
    c = lax.axis_index("c")
    s = lax.axis_index("s")
    wid = c * NS + s

    pltpu.sync_copy(zf_hbm.at[pl.ds(s * RPT, RPT)], acc_sh.at[pl.ds(s * RPT, RPT)])
    plsc.subcore_barrier()

    def body(j, carry):
        g = (wid * NCHUNK + j) * CH
        pltpu.sync_copy(src_hbm.at[pl.ds(g, CH)], src_f)
        pltpu.sync_copy(dst_hbm.at[pl.ds(g, CH)], dst_f)
        pltpu.async_copy(x_hbm.at[src_f], rows_v, sem).wait()
        pltpu.sync_copy(rows_v, acc_sh.at[dst_f], add=True)
        return carry

    lax.fori_loop(0, NCHUNK, body, 0)
    plsc.subcore_barrier()

    pltpu.sync_copy(acc_sh.at[pl.ds(s * RPT, RPT)],
                    sums_out.at[c, pl.ds(s * RPT, RPT)])


B = 1024


def _dense_body(relu, p_ref, c_ref, t_ref, wl_ref, b_ref, wr_ref, o_ref):
    ssum = p_ref[0] + p_ref[1]
    cnt = c_ref[0, :, 0:1] + c_ref[1, :, 0:1]
    mean = ssum / jnp.maximum(cnt, 1.0)
    r = (lax.dot_general(mean, wl_ref[...], (((1,), (1,)), ((), ())),
                         preferred_element_type=jnp.float32)
         + b_ref[...]
         + lax.dot_general(t_ref[...], wr_ref[...], (((1,), (1,)), ((), ())),
                           preferred_element_type=jnp.float32))
    o_ref[...] = jnp.maximum(r, 0.0) if relu else r


def _dense(p, cnts, t, W_l, b, W_r, relu):
    out_f = W_l.shape[0]
    return pl.pallas_call(
        functools.partial(_dense_body, relu),
        grid=(NPAD // B,),
        in_specs=[
            pl.BlockSpec((NC, B, F), lambda i: (0, i, 0)),
            pl.BlockSpec((NC, B, F), lambda i: (0, i, 0)),
            pl.BlockSpec((B, F), lambda i: (i, 0)),
            pl.BlockSpec((out_f, F), lambda i: (0, 0)),
            pl.BlockSpec((1, out_f), lambda i: (0, 0)),
            pl.BlockSpec((out_f, F), lambda i: (0, 0)),
        ],
        out_specs=pl.BlockSpec((B, out_f), lambda i: (i, 0)),
        out_shape=jax.ShapeDtypeStruct((NPAD, out_f), jnp.float32),
    )(p, cnts, t, W_l, b.reshape(1, out_f), W_r)


def kernel(x, edge_index, W1_l, b1, W1_r, W2_l, b2, W2_r):
    src = edge_index[0]
    dst = edge_index[1]
    zf = jnp.zeros((NPAD, F), jnp.float32)
    ones_tab = jnp.ones((NPAD, F), jnp.float32)
    xp = jnp.pad(x, ((0, NPAD - N), (0, 0)))

    cnts, = _sc_agg(ones_tab, src, dst, zf)
    xp_seq, src_seq = lax.optimization_barrier((xp, src, cnts))[:2]
    sums1, = _sc_agg(xp_seq, src_seq, dst, zf)
    h = _dense(sums1, cnts, xp, W1_l, b1, W1_r, relu=True)
    sums2, = _sc_agg(h, src, dst, zf)
    out = _dense(sums2, cnts, h, W2_l, b2, W2_r, relu=False)
    return out[:N]


# speedup vs baseline: 6.1902x; 1.6074x over previous
import functools

import jax
import jax.numpy as jnp
from jax import lax
from jax.experimental import pallas as pl
from jax.experimental.pallas import tpu as pltpu
from jax.experimental.pallas import tpu_sc as plsc

N = 10000
E = 320000
F = 128
OUT_F = 121

NC = 2
NS = 16
NW = NC * NS
EPW = E // NW
CH = 80
NCHUNK = EPW // CH
G = NW * NCHUNK
NPAD = 10240
RPT = NPAD // NS
CW = 16

_mesh = plsc.VectorSubcoreMesh(core_axis_name="c", subcore_axis_name="s")


@functools.partial(
    pl.kernel,
    mesh=_mesh,
    out_type=[
        jax.ShapeDtypeStruct((NC, NPAD, F), jnp.float32),
    ],
    scratch_types=[
        pltpu.VMEM((CH,), jnp.int32),
        pltpu.VMEM((CH,), jnp.int32),
        pltpu.VMEM((CH,), jnp.int32),
        pltpu.VMEM((CH,), jnp.int32),
        pltpu.VMEM((CH, F), jnp.float32),
        pltpu.VMEM((CH, F), jnp.float32),
        pltpu.VMEM_SHARED((NPAD, F), jnp.float32),
        pltpu.SemaphoreType.DMA,
        pltpu.SemaphoreType.DMA,
    ],
)
def _sc_agg(x_hbm, src_hbm, dst_hbm, zf_hbm,
            sums_out,
            src0, dst0, src1, dst1, rows0, rows1, acc_sh, sem0, sem1):
    c = lax.axis_index("c")
    s = lax.axis_index("s")
    wid = c * NS + s

    pltpu.sync_copy(zf_hbm.at[pl.ds(s * RPT, RPT)], acc_sh.at[pl.ds(s * RPT, RPT)])
    plsc.subcore_barrier()

    base = wid * NCHUNK

    def load_idx(j, src_f, dst_f):
        g = jnp.minimum(base + j, base + NCHUNK - 1) * CH
        pltpu.sync_copy(src_hbm.at[pl.ds(g, CH)], src_f)
        pltpu.sync_copy(dst_hbm.at[pl.ds(g, CH)], dst_f)

    load_idx(0, src0, dst0)
    pltpu.async_copy(x_hbm.at[src0], rows0, sem0)
    load_idx(1, src1, dst1)

    def body(i, carry):
        j0 = 2 * i
        pltpu.async_copy(x_hbm.at[src1], rows1, sem1)
        pltpu.make_async_copy(x_hbm.at[src0], rows0, sem0).wait()
        pltpu.sync_copy(rows0, acc_sh.at[dst0], add=True)
        load_idx(j0 + 2, src0, dst0)
        pltpu.async_copy(x_hbm.at[src0], rows0, sem0)
        pltpu.make_async_copy(x_hbm.at[src1], rows1, sem1).wait()
        pltpu.sync_copy(rows1, acc_sh.at[dst1], add=True)
        load_idx(j0 + 3, src1, dst1)
        return carry

    lax.fori_loop(0, (NCHUNK - 1) // 2, body, 0)
    pltpu.make_async_copy(x_hbm.at[src0], rows0, sem0).wait()
    pltpu.sync_copy(rows0, acc_sh.at[dst0], add=True)
    plsc.subcore_barrier()

    pltpu.sync_copy(acc_sh.at[pl.ds(s * RPT, RPT)],
                    sums_out.at[c, pl.ds(s * RPT, RPT)])


B = 1024


def _dense_body(relu, p_ref, c_ref, t_ref, wl_ref, b_ref, wr_ref, o_ref):
    ssum = p_ref[0] + p_ref[1]
    cnt = c_ref[0, :, 0:1] + c_ref[1, :, 0:1]
    mean = ssum / jnp.maximum(cnt, 1.0)
    r = (lax.dot_general(mean, wl_ref[...], (((1,), (1,)), ((), ())),
                         preferred_element_type=jnp.float32)
         + b_ref[...]
         + lax.dot_general(t_ref[...], wr_ref[...], (((1,), (1,)), ((), ())),
                           preferred_element_type=jnp.float32))
    o_ref[...] = jnp.maximum(r, 0.0) if relu else r


def _dense(p, cnts, t, W_l, b, W_r, relu):
    out_f = W_l.shape[0]
    return pl.pallas_call(
        functools.partial(_dense_body, relu),
        grid=(NPAD // B,),
        in_specs=[
            pl.BlockSpec((NC, B, F), lambda i: (0, i, 0)),
            pl.BlockSpec((NC, B, F), lambda i: (0, i, 0)),
            pl.BlockSpec((B, F), lambda i: (i, 0)),
            pl.BlockSpec((out_f, F), lambda i: (0, 0)),
            pl.BlockSpec((1, out_f), lambda i: (0, 0)),
            pl.BlockSpec((out_f, F), lambda i: (0, 0)),
        ],
        out_specs=pl.BlockSpec((B, out_f), lambda i: (i, 0)),
        out_shape=jax.ShapeDtypeStruct((NPAD, out_f), jnp.float32),
    )(p, cnts, t, W_l, b.reshape(1, out_f), W_r)


def kernel(x, edge_index, W1_l, b1, W1_r, W2_l, b2, W2_r):
    src = edge_index[0]
    dst = edge_index[1]
    zf = jnp.zeros((NPAD, F), jnp.float32)
    ones_tab = jnp.ones((NPAD, F), jnp.float32)
    xp = jnp.pad(x, ((0, NPAD - N), (0, 0)))

    cnts, = _sc_agg(ones_tab, src, dst, zf)
    xp_seq, src_seq = lax.optimization_barrier((xp, src, cnts))[:2]
    sums1, = _sc_agg(xp_seq, src_seq, dst, zf)
    h = _dense(sums1, cnts, xp, W1_l, b1, W1_r, relu=True)
    sums2, = _sc_agg(h, src, dst, zf)
    out = _dense(sums2, cnts, h, W2_l, b2, W2_r, relu=False)
    return out[:N]
